# Initial kernel scaffold; baseline (speedup 1.0000x reference)
import jax
import jax.numpy as jnp
from jax.experimental import pallas as pl


def kernel(xyzts, table):
    raise NotImplementedError("write your pallas kernel here")


# speedup vs baseline: 4.6307x; 4.6307x over previous
import functools

import jax
import jax.numpy as jnp
import numpy as np
from jax import lax
from jax.experimental import pallas as pl
from jax.experimental.pallas import tpu as pltpu
from jax.experimental.pallas import tpu_sc as plsc

_B = 131072
_NUM_SCALES = 16
_MIN_RES = np.array([16.0, 16.0, 16.0, 4.0])
_MAX_RES = np.array([512.0, 512.0, 512.0, 32.0])
_MAX_PARAMS = 2 ** 19

_NC, _NS, _L = 2, 16, 16
_NW = _NC * _NS
_BW = _B // _NW
_C = 256
_NCHUNK = _BW // _C
_NSL = _C // _L

_MASK = _MAX_PARAMS - 1
_P1 = int(np.uint32(2654435761).view(np.int32))
_P2 = int(np.uint32(805459861).view(np.int32))
_P3 = int(np.uint32(3674653429).view(np.int32))


def _levels():
    g = np.exp((np.log(_MAX_RES) - np.log(_MIN_RES)) / (_NUM_SCALES - 1))
    out = []
    off = 0
    for lvl in range(_NUM_SCALES):
        res = np.ceil(_MIN_RES * (g ** lvl)).astype(np.int64)
        full = int(np.prod(res + 1))
        dense = full <= _MAX_PARAMS
        size = full if dense else _MAX_PARAMS
        s1 = int(res[0] + 1)
        s2 = s1 * int(res[1] + 1)
        s3 = s2 * int(res[2] + 1)
        out.append(dict(res=[float(r) for r in res],
                        resm1=[int(r) - 1 for r in res],
                        dense=dense, mul=(s1, s2, s3) if dense
                        else (_P1, _P2, _P3),
                        off=off // 2))
        off += size * 2
    return out


_LVL = _levels()


def _body(xyzt_hbm, table_hbm, out_hbm,
          coords_v, out_v,
          idx0_v, idx1_v, lo0_v, lo1_v, w0_v, w1_v, rows0_v, rows1_v,
          sem0, sem1):
    cid = lax.axis_index("c")
    sid = lax.axis_index("s")
    wid = sid * _NC + cid
    base = wid * _BW

    iota = lax.iota(jnp.int32, _L)
    idx_b = (idx0_v, idx1_v)
    lo_b = (lo0_v, lo1_v)
    w_b = (w0_v, w1_v)
    rows_b = (rows0_v, rows1_v)
    sem_b = (sem0, sem1)

    def phase_a(lvl, b):
        p = _LVL[lvl]
        idx_v, lo_v, wbuf_v = idx_b[b], lo_b[b], w_b[b]
        res = [jnp.full((_L,), r, jnp.float32) for r in p["res"]]
        resm1 = [jnp.full((_L,), r, jnp.int32) for r in p["resm1"]]
        mul = [jnp.full((_L,), m, jnp.int32) for m in p["mul"]]
        off = p["off"]
        dense = p["dense"]

        def slice_body(s, _):
            o16 = s * _L
            gi, fr = [], []
            for d in range(4):
                pos = coords_v[d, pl.ds(o16, _L)] * res[d]
                g = jnp.minimum(pos.astype(jnp.int32), resm1[d])
                gi.append(g)
                fr.append(jnp.clip(pos - g.astype(jnp.float32), 0.0, 1.0))
            w0 = [1.0 - f for f in fr]
            wxy = [w0[0] * w0[1], fr[0] * w0[1], w0[0] * fr[1], fr[0] * fr[1]]
            wzt = [w0[2] * w0[3], fr[2] * w0[3], w0[2] * fr[3], fr[2] * fr[3]]
            hx = [gi[0], gi[0] + 1]
            hy0 = gi[1] * mul[0]
            hz0 = gi[2] * mul[1]
            ht0 = gi[3] * mul[2]
            hy = [hy0, hy0 + mul[0]]
            hz = [hz0, hz0 + mul[1]]
            ht = [ht0, ht0 + mul[2]]
            if dense:
                axy = [hx[i] + hy[j] for j in (0, 1) for i in (0, 1)]
                bzt = [hz[i] + ht[j] + off for j in (0, 1) for i in (0, 1)]
            else:
                axy = [hx[i] ^ hy[j] for j in (0, 1) for i in (0, 1)]
                bzt = [hz[i] ^ ht[j] for j in (0, 1) for i in (0, 1)]
            for c in range(16):
                w = wxy[c & 3] * wzt[c >> 2]
                wbuf_v[c, pl.ds(o16, _L)] = w
                if dense:
                    row = axy[c & 3] + bzt[c >> 2]
                else:
                    row = ((axy[c & 3] ^ bzt[c >> 2]) & _MASK) + off
                flat = s * 256 + c * _L
                idx_v[pl.ds(flat, _L)] = lax.shift_right_logical(row, 2)
                lo_v[pl.ds(flat, _L)] = (row & 3) * 2
            return _

        lax.fori_loop(0, _NSL, slice_body, None)

    def fire(b):
        return pltpu.async_copy(table_hbm.at[idx_b[b]], rows_b[b], sem_b[b])

    def phase_c(lvl, b):
        lo_v, wbuf_v, rows_v = lo_b[b], w_b[b], rows_b[b]

        def slice_body(s, _):
            o16 = s * _L
            f0 = jnp.zeros((_L,), jnp.float32)
            f1 = jnp.zeros((_L,), jnp.float32)
            for c in range(16):
                flat = s * 256 + c * _L
                rr = iota + flat
                lo = lo_v[pl.ds(flat, _L)]
                r0 = plsc.load_gather(rows_v, [rr, lo])
                r1 = plsc.load_gather(rows_v, [rr, lo + 1])
                w = wbuf_v[c, pl.ds(o16, _L)]
                f0 = f0 + w * r0
                f1 = f1 + w * r1
            out_v[2 * lvl, pl.ds(o16, _L)] = f0
            out_v[2 * lvl + 1, pl.ds(o16, _L)] = f1
            return _

        lax.fori_loop(0, _NSL, slice_body, None)

    def chunk_body(chunk, _):
        c0 = base + chunk * _C
        pltpu.sync_copy(xyzt_hbm.at[:, pl.ds(c0, _C)], coords_v)
        pending = None
        for lvl in range(_NUM_SCALES):
            b = lvl & 1
            phase_a(lvl, b)
            d = fire(b)
            if pending is not None:
                pending.wait()
                phase_c(lvl - 1, 1 - b)
            pending = d
        pending.wait()
        phase_c(_NUM_SCALES - 1, 1)
        pltpu.sync_copy(out_v, out_hbm.at[wid, :, pl.ds(chunk * _C, _C)])
        return _

    lax.fori_loop(0, _NCHUNK, chunk_body, None)


@functools.cache
def _make_enc():
    return pl.kernel(
        _body,
        out_type=jax.ShapeDtypeStruct((_NW, 2 * _NUM_SCALES, _BW),
                                      jnp.float32),
        mesh=plsc.VectorSubcoreMesh(core_axis_name="c", subcore_axis_name="s",
                                    num_cores=_NC, num_subcores=_NS),
        compiler_params=pltpu.CompilerParams(needs_layout_passes=False,
                                             use_tc_tiling_on_sc=False),
        scratch_types=[
            pltpu.VMEM((4, _C), jnp.float32),
            pltpu.VMEM((2 * _NUM_SCALES, _C), jnp.float32),
            pltpu.VMEM((16 * _C,), jnp.int32),
            pltpu.VMEM((16 * _C,), jnp.int32),
            pltpu.VMEM((16 * _C,), jnp.int32),
            pltpu.VMEM((16 * _C,), jnp.int32),
            pltpu.VMEM((16, _C), jnp.float32),
            pltpu.VMEM((16, _C), jnp.float32),
            pltpu.VMEM((16 * _C, 8), jnp.float32),
            pltpu.VMEM((16 * _C, 8), jnp.float32),
            pltpu.SemaphoreType.DMA,
            pltpu.SemaphoreType.DMA,
        ],
    )


@jax.jit
def kernel(xyzts, table):
    xyzt_t = xyzts.T
    table8 = jnp.pad(table, (0, 4)).reshape(-1, 8)
    out = _make_enc()(xyzt_t, table8)
    return out.transpose(0, 2, 1).reshape(_B, 2 * _NUM_SCALES)


# speedup vs baseline: 4.6667x; 1.0078x over previous
import functools

import jax
import jax.numpy as jnp
import numpy as np
from jax import lax
from jax.experimental import pallas as pl
from jax.experimental.pallas import tpu as pltpu
from jax.experimental.pallas import tpu_sc as plsc

_B = 131072
_NUM_SCALES = 16
_MIN_RES = np.array([16.0, 16.0, 16.0, 4.0])
_MAX_RES = np.array([512.0, 512.0, 512.0, 32.0])
_MAX_PARAMS = 2 ** 19

_NC, _NS, _L = 2, 16, 16
_NW = _NC * _NS
_BW = _B // _NW
_C = 256
_NCHUNK = _BW // _C
_NSL = _C // _L

_MASK = _MAX_PARAMS - 1
_P1 = int(np.uint32(2654435761).view(np.int32))
_P2 = int(np.uint32(805459861).view(np.int32))
_P3 = int(np.uint32(3674653429).view(np.int32))


def _levels():
    g = np.exp((np.log(_MAX_RES) - np.log(_MIN_RES)) / (_NUM_SCALES - 1))
    out = []
    off = 0
    for lvl in range(_NUM_SCALES):
        res = np.ceil(_MIN_RES * (g ** lvl)).astype(np.int64)
        full = int(np.prod(res + 1))
        dense = full <= _MAX_PARAMS
        size = full if dense else _MAX_PARAMS
        s1 = int(res[0] + 1)
        s2 = s1 * int(res[1] + 1)
        s3 = s2 * int(res[2] + 1)
        out.append(dict(res=[float(r) for r in res],
                        resm1=[int(r) - 1 for r in res],
                        dense=dense, mul=(s1, s2, s3) if dense
                        else (_P1, _P2, _P3),
                        off=off // 2))
        off += size * 2
    return out


_LVL = _levels()


def _body(xyzt_hbm, table_hbm, out_hbm,
          coords_v, out_v,
          idx0_v, idx1_v, lo0_v, lo1_v, w0_v, w1_v, rows0_v, rows1_v,
          sem0a, sem0b, sem1a, sem1b):
    cid = lax.axis_index("c")
    sid = lax.axis_index("s")
    wid = sid * _NC + cid
    base = wid * _BW

    iota = lax.iota(jnp.int32, _L)
    idx_b = (idx0_v, idx1_v)
    lo_b = (lo0_v, lo1_v)
    w_b = (w0_v, w1_v)
    rows_b = (rows0_v, rows1_v)
    sem_b = ((sem0a, sem0b), (sem1a, sem1b))

    def phase_a(lvl, b):
        p = _LVL[lvl]
        idx_v, lo_v, wbuf_v = idx_b[b], lo_b[b], w_b[b]
        res = [jnp.full((_L,), r, jnp.float32) for r in p["res"]]
        resm1 = [jnp.full((_L,), r, jnp.int32) for r in p["resm1"]]
        mul = [jnp.full((_L,), m, jnp.int32) for m in p["mul"]]
        off = p["off"]
        dense = p["dense"]

        def slice_body(s, _):
            o16 = s * _L
            gi, fr = [], []
            for d in range(4):
                pos = coords_v[d, pl.ds(o16, _L)] * res[d]
                g = jnp.minimum(pos.astype(jnp.int32), resm1[d])
                gi.append(g)
                fr.append(jnp.clip(pos - g.astype(jnp.float32), 0.0, 1.0))
            w0 = [1.0 - f for f in fr]
            wxy = [w0[0] * w0[1], fr[0] * w0[1], w0[0] * fr[1], fr[0] * fr[1]]
            wzt = [w0[2] * w0[3], fr[2] * w0[3], w0[2] * fr[3], fr[2] * fr[3]]
            hx = [gi[0], gi[0] + 1]
            hy0 = gi[1] * mul[0]
            hz0 = gi[2] * mul[1]
            ht0 = gi[3] * mul[2]
            hy = [hy0, hy0 + mul[0]]
            hz = [hz0, hz0 + mul[1]]
            ht = [ht0, ht0 + mul[2]]
            if dense:
                axy = [hx[i] + hy[j] for j in (0, 1) for i in (0, 1)]
                bzt = [hz[i] + ht[j] + off for j in (0, 1) for i in (0, 1)]
            else:
                axy = [hx[i] ^ hy[j] for j in (0, 1) for i in (0, 1)]
                bzt = [hz[i] ^ ht[j] for j in (0, 1) for i in (0, 1)]
            for c in range(16):
                w = wxy[c & 3] * wzt[c >> 2]
                wbuf_v[c, pl.ds(o16, _L)] = w
                if dense:
                    row = axy[c & 3] + bzt[c >> 2]
                else:
                    row = ((axy[c & 3] ^ bzt[c >> 2]) & _MASK) + off
                flat = s * 256 + c * _L
                idx_v[pl.ds(flat, _L)] = lax.shift_right_logical(row, 2)
                lo_v[pl.ds(flat, _L)] = (row & 3) * 2
            return _

        lax.fori_loop(0, _NSL, slice_body, None)

    def fire(b):
        h = 16 * _C // 2
        sa, sb = sem_b[b]
        d1 = pltpu.async_copy(table_hbm.at[idx_b[b].at[pl.ds(0, h)]],
                              rows_b[b].at[pl.ds(0, h)], sa)
        d2 = pltpu.async_copy(table_hbm.at[idx_b[b].at[pl.ds(h, h)]],
                              rows_b[b].at[pl.ds(h, h)], sb)
        return d1, d2

    def phase_c(lvl, b):
        lo_v, wbuf_v, rows_v = lo_b[b], w_b[b], rows_b[b]

        def slice_body(s, _):
            o16 = s * _L
            f0 = jnp.zeros((_L,), jnp.float32)
            f1 = jnp.zeros((_L,), jnp.float32)
            for c in range(16):
                flat = s * 256 + c * _L
                rr = iota + flat
                lo = lo_v[pl.ds(flat, _L)]
                r0 = plsc.load_gather(rows_v, [rr, lo])
                r1 = plsc.load_gather(rows_v, [rr, lo + 1])
                w = wbuf_v[c, pl.ds(o16, _L)]
                f0 = f0 + w * r0
                f1 = f1 + w * r1
            out_v[2 * lvl, pl.ds(o16, _L)] = f0
            out_v[2 * lvl + 1, pl.ds(o16, _L)] = f1
            return _

        lax.fori_loop(0, _NSL, slice_body, None)

    def chunk_body(chunk, _):
        c0 = base + chunk * _C
        pltpu.sync_copy(xyzt_hbm.at[:, pl.ds(c0, _C)], coords_v)
        pending = None
        for lvl in range(_NUM_SCALES):
            b = lvl & 1
            phase_a(lvl, b)
            d = fire(b)
            if pending is not None:
                pending[0].wait()
                pending[1].wait()
                phase_c(lvl - 1, 1 - b)
            pending = d
        pending[0].wait()
        pending[1].wait()
        phase_c(_NUM_SCALES - 1, 1)
        pltpu.sync_copy(out_v, out_hbm.at[wid, :, pl.ds(chunk * _C, _C)])
        return _

    lax.fori_loop(0, _NCHUNK, chunk_body, None)


@functools.cache
def _make_enc():
    return pl.kernel(
        _body,
        out_type=jax.ShapeDtypeStruct((_NW, 2 * _NUM_SCALES, _BW),
                                      jnp.float32),
        mesh=plsc.VectorSubcoreMesh(core_axis_name="c", subcore_axis_name="s",
                                    num_cores=_NC, num_subcores=_NS),
        compiler_params=pltpu.CompilerParams(needs_layout_passes=False,
                                             use_tc_tiling_on_sc=False),
        scratch_types=[
            pltpu.VMEM((4, _C), jnp.float32),
            pltpu.VMEM((2 * _NUM_SCALES, _C), jnp.float32),
            pltpu.VMEM((16 * _C,), jnp.int32),
            pltpu.VMEM((16 * _C,), jnp.int32),
            pltpu.VMEM((16 * _C,), jnp.int32),
            pltpu.VMEM((16 * _C,), jnp.int32),
            pltpu.VMEM((16, _C), jnp.float32),
            pltpu.VMEM((16, _C), jnp.float32),
            pltpu.VMEM((16 * _C, 8), jnp.float32),
            pltpu.VMEM((16 * _C, 8), jnp.float32),
            pltpu.SemaphoreType.DMA,
            pltpu.SemaphoreType.DMA,
            pltpu.SemaphoreType.DMA,
            pltpu.SemaphoreType.DMA,
        ],
    )


@jax.jit
def kernel(xyzts, table):
    xyzt_t = xyzts.T
    table8 = jnp.pad(table, (0, 4)).reshape(-1, 8)
    out = _make_enc()(xyzt_t, table8)
    return out.transpose(0, 2, 1).reshape(_B, 2 * _NUM_SCALES)


# speedup vs baseline: 4.6810x; 1.0031x over previous
import functools

import jax
import jax.numpy as jnp
import numpy as np
from jax import lax
from jax.experimental import pallas as pl
from jax.experimental.pallas import tpu as pltpu
from jax.experimental.pallas import tpu_sc as plsc

_B = 131072
_NUM_SCALES = 16
_MIN_RES = np.array([16.0, 16.0, 16.0, 4.0])
_MAX_RES = np.array([512.0, 512.0, 512.0, 32.0])
_MAX_PARAMS = 2 ** 19

_NC, _NS, _L = 2, 16, 16
_NW = _NC * _NS
_BW = _B // _NW
_C = 256
_NCHUNK = _BW // _C
_NSL = _C // _L

_MASK = _MAX_PARAMS - 1
_P1 = int(np.uint32(2654435761).view(np.int32))
_P2 = int(np.uint32(805459861).view(np.int32))
_P3 = int(np.uint32(3674653429).view(np.int32))


def _levels():
    g = np.exp((np.log(_MAX_RES) - np.log(_MIN_RES)) / (_NUM_SCALES - 1))
    out = []
    off = 0
    for lvl in range(_NUM_SCALES):
        res = np.ceil(_MIN_RES * (g ** lvl)).astype(np.int64)
        full = int(np.prod(res + 1))
        dense = full <= _MAX_PARAMS
        size = full if dense else _MAX_PARAMS
        s1 = int(res[0] + 1)
        s2 = s1 * int(res[1] + 1)
        s3 = s2 * int(res[2] + 1)
        out.append(dict(res=[float(r) for r in res],
                        resm1=[int(r) - 1 for r in res],
                        dense=dense, mul=(s1, s2, s3) if dense
                        else (_P1, _P2, _P3),
                        off=off // 2))
        off += size * 2
    return out


_LVL = _levels()


def _body(xyzt_hbm, table_hbm, out_hbm,
          coords_v, out_v,
          idx0_v, idx1_v, lo0_v, lo1_v, rows0_v, rows1_v,
          sem0a, sem0b, sem1a, sem1b):
    cid = lax.axis_index("c")
    sid = lax.axis_index("s")
    wid = sid * _NC + cid
    base = wid * _BW

    iota = lax.iota(jnp.int32, _L)
    idx_b = (idx0_v, idx1_v)
    lo_b = (lo0_v, lo1_v)
    rows_b = (rows0_v, rows1_v)
    sem_b = ((sem0a, sem0b), (sem1a, sem1b))

    def grid_coords(s, p, want_frac):
        res = [jnp.full((_L,), r, jnp.float32) for r in p["res"]]
        resm1 = [jnp.full((_L,), r, jnp.int32) for r in p["resm1"]]
        o16 = s * _L
        gi, fr = [], []
        for d in range(4):
            pos = coords_v[d, pl.ds(o16, _L)] * res[d]
            g = jnp.minimum(pos.astype(jnp.int32), resm1[d])
            gi.append(g)
            if want_frac:
                fr.append(jnp.minimum(pos - g.astype(jnp.float32), 1.0))
        return gi, fr

    def phase_a(lvl, b):
        p = _LVL[lvl]
        idx_v, lo_v = idx_b[b], lo_b[b]
        mul = [jnp.full((_L,), m, jnp.int32) for m in p["mul"]]
        off = p["off"]
        dense = p["dense"]

        def slice_body(s, _):
            gi, _fr = grid_coords(s, p, False)
            hx = [gi[0], gi[0] + 1]
            hy0 = gi[1] * mul[0]
            hz0 = gi[2] * mul[1]
            ht0 = gi[3] * mul[2]
            hy = [hy0, hy0 + mul[0]]
            hz = [hz0, hz0 + mul[1]]
            ht = [ht0, ht0 + mul[2]]
            if dense:
                axy = [hx[i] + hy[j] for j in (0, 1) for i in (0, 1)]
                bzt = [hz[i] + ht[j] + off for j in (0, 1) for i in (0, 1)]
            else:
                axy = [hx[i] ^ hy[j] for j in (0, 1) for i in (0, 1)]
                bzt = [hz[i] ^ ht[j] for j in (0, 1) for i in (0, 1)]
            for c in range(16):
                if dense:
                    row = axy[c & 3] + bzt[c >> 2]
                else:
                    row = ((axy[c & 3] ^ bzt[c >> 2]) & _MASK) + off
                flat = s * 256 + c * _L
                idx_v[pl.ds(flat, _L)] = lax.shift_right_logical(row, 2)
                lo_v[pl.ds(flat, _L)] = (row & 3) * 2
            return _

        lax.fori_loop(0, _NSL, slice_body, None)

    def fire(b):
        h = 16 * _C // 2
        sa, sb = sem_b[b]
        d1 = pltpu.async_copy(table_hbm.at[idx_b[b].at[pl.ds(0, h)]],
                              rows_b[b].at[pl.ds(0, h)], sa)
        d2 = pltpu.async_copy(table_hbm.at[idx_b[b].at[pl.ds(h, h)]],
                              rows_b[b].at[pl.ds(h, h)], sb)
        return d1, d2

    def phase_c(lvl, b):
        p = _LVL[lvl]
        lo_v, rows_v = lo_b[b], rows_b[b]

        def slice_body(s, _):
            o16 = s * _L
            _gi, fr = grid_coords(s, p, True)
            w0 = [1.0 - f for f in fr]
            wxy = [w0[0] * w0[1], fr[0] * w0[1], w0[0] * fr[1], fr[0] * fr[1]]
            wzt = [w0[2] * w0[3], fr[2] * w0[3], w0[2] * fr[3], fr[2] * fr[3]]
            f0 = jnp.zeros((_L,), jnp.float32)
            f1 = jnp.zeros((_L,), jnp.float32)
            for c in range(16):
                flat = s * 256 + c * _L
                rr = iota + flat
                lo = lo_v[pl.ds(flat, _L)]
                r0 = plsc.load_gather(rows_v, [rr, lo])
                r1 = plsc.load_gather(rows_v, [rr, lo + 1])
                w = wxy[c & 3] * wzt[c >> 2]
                f0 = f0 + w * r0
                f1 = f1 + w * r1
            out_v[2 * lvl, pl.ds(o16, _L)] = f0
            out_v[2 * lvl + 1, pl.ds(o16, _L)] = f1
            return _

        lax.fori_loop(0, _NSL, slice_body, None)

    def chunk_body(chunk, _):
        c0 = base + chunk * _C
        pltpu.sync_copy(xyzt_hbm.at[:, pl.ds(c0, _C)], coords_v)
        pending = None
        for lvl in range(_NUM_SCALES):
            b = lvl & 1
            phase_a(lvl, b)
            d = fire(b)
            if pending is not None:
                pending[0].wait()
                pending[1].wait()
                phase_c(lvl - 1, 1 - b)
            pending = d
        pending[0].wait()
        pending[1].wait()
        phase_c(_NUM_SCALES - 1, 1)
        pltpu.sync_copy(out_v, out_hbm.at[wid, :, pl.ds(chunk * _C, _C)])
        return _

    lax.fori_loop(0, _NCHUNK, chunk_body, None)


@functools.cache
def _make_enc():
    return pl.kernel(
        _body,
        out_type=jax.ShapeDtypeStruct((_NW, 2 * _NUM_SCALES, _BW),
                                      jnp.float32),
        mesh=plsc.VectorSubcoreMesh(core_axis_name="c", subcore_axis_name="s",
                                    num_cores=_NC, num_subcores=_NS),
        compiler_params=pltpu.CompilerParams(needs_layout_passes=False,
                                             use_tc_tiling_on_sc=False),
        scratch_types=[
            pltpu.VMEM((4, _C), jnp.float32),
            pltpu.VMEM((2 * _NUM_SCALES, _C), jnp.float32),
            pltpu.VMEM((16 * _C,), jnp.int32),
            pltpu.VMEM((16 * _C,), jnp.int32),
            pltpu.VMEM((16 * _C,), jnp.int32),
            pltpu.VMEM((16 * _C,), jnp.int32),
            pltpu.VMEM((16 * _C, 8), jnp.float32),
            pltpu.VMEM((16 * _C, 8), jnp.float32),
            pltpu.SemaphoreType.DMA,
            pltpu.SemaphoreType.DMA,
            pltpu.SemaphoreType.DMA,
            pltpu.SemaphoreType.DMA,
        ],
    )


@jax.jit
def kernel(xyzts, table):
    xyzt_t = xyzts.T
    table8 = jnp.pad(table, (0, 4)).reshape(-1, 8)
    out = _make_enc()(xyzt_t, table8)
    return out.transpose(0, 2, 1).reshape(_B, 2 * _NUM_SCALES)
